# baseline (speedup 1.0000x reference)
import functools

import jax
import jax.numpy as jnp
from jax import lax
from jax.experimental import pallas as pl
from jax.experimental.pallas import tpu as pltpu
from jax.experimental.pallas import tpu_sc as plsc

EPS = 1e-12

_NC = 2
_NS = 16
_NW = _NC * _NS

_CHUNK = 32
_NBUF = 4
_NSPLIT = 4


def _sc_gather_body(n_tokens, hidden, nbuf, ids_hbm, table_hbm, out_hbm, idx_v,
                    *bufs_and_sems):
    rows = bufs_and_sems[:nbuf]
    gsem = bufs_and_sems[nbuf:2 * nbuf]
    wsem = bufs_and_sems[2 * nbuf:]
    tok_per_w = n_tokens // _NW
    n_chunks = tok_per_w // _CHUNK
    wid = lax.axis_index("s") * _NC + lax.axis_index("c")
    base = wid * tok_per_w
    pltpu.sync_copy(ids_hbm.at[pl.ds(base, tok_per_w)], idx_v)

    def gather(c, b):
        return pltpu.async_copy(
            table_hbm.at[idx_v.at[pl.ds(c * _CHUNK, _CHUNK)]], rows[b], gsem[b])

    def writeback(c, b):
        return pltpu.async_copy(
            rows[b], out_hbm.at[pl.ds(base + c * _CHUNK, _CHUNK)], wsem[b])

    copies = [None] * nbuf
    wbs = [None] * nbuf
    for c in range(min(nbuf, n_chunks)):
        copies[c] = gather(c, c)
    for c in range(n_chunks):
        b = c % nbuf
        copies[b].wait()
        wbs[b] = writeback(c, b)
        nxt = c + nbuf
        if nxt < n_chunks:
            wbs[b].wait()
            copies[b] = gather(nxt, b)
    for c in range(max(0, n_chunks - nbuf), n_chunks):
        wbs[c % nbuf].wait()


def _sc_gather(ids_flat, table):
    n_tokens = ids_flat.shape[0]
    hidden = table.shape[1]
    tok_per_w = n_tokens // _NW
    nbuf = min(_NBUF, tok_per_w // _CHUNK)
    mesh = plsc.VectorSubcoreMesh(core_axis_name="c", subcore_axis_name="s")
    call = pl.kernel(
        functools.partial(_sc_gather_body, n_tokens, hidden, nbuf),
        mesh=mesh,
        out_type=jax.ShapeDtypeStruct((n_tokens, hidden), jnp.float32),
        scratch_types=[
            pltpu.VMEM((tok_per_w,), jnp.int32),
        ] + [pltpu.VMEM((_CHUNK, hidden), jnp.float32) for _ in range(nbuf)]
          + [pltpu.SemaphoreType.DMA for _ in range(2 * nbuf)],
    )
    return call(ids_flat, table)


def _tc_ln_body(x_ref, pos_ref, tt_ref, t_ref, w_ref, b_ref, o_ref):
    x = x_ref[...]
    tt0 = tt_ref[0:1, :]
    dtt = tt_ref[1:2, :] - tt0
    x = x + pos_ref[...] + tt0 + t_ref[...] * dtt
    u = jnp.mean(x, axis=-1, keepdims=True)
    v = jnp.mean((x - u) ** 2, axis=-1, keepdims=True)
    y = (x - u) * lax.rsqrt(v + EPS)
    o_ref[...] = y * w_ref[...] + b_ref[...]


def _tc_ln_body_prev(prev_ref, x_ref, pos_ref, tt_ref, t_ref, w_ref, b_ref, o_ref):
    del prev_ref
    _tc_ln_body(x_ref, pos_ref, tt_ref, t_ref, w_ref, b_ref, o_ref)


def _tc_ln_slice(prev, g_s, pos_emb, tt_emb, t_f32, lnw, lnb, s_idx, blk,
                 n_tokens, seq):
    hidden = g_s.shape[1]
    batch = n_tokens // seq
    n_sb = seq // blk
    grid = (batch,)
    specs = [
        pl.BlockSpec((blk, hidden), lambda b: (b, 0)),
        pl.BlockSpec((blk, hidden), lambda b: (s_idx, 0)),
        pl.BlockSpec((2, hidden), lambda b: (0, 0)),
        pl.BlockSpec((blk, 1), lambda b: (b * n_sb + s_idx, 0)),
        pl.BlockSpec((1, hidden), lambda b: (0, 0)),
        pl.BlockSpec((1, hidden), lambda b: (0, 0)),
    ]
    out_spec = pl.BlockSpec((blk, hidden), lambda b: (b * n_sb + s_idx, 0))
    out_shape = jax.ShapeDtypeStruct((n_tokens, hidden), jnp.float32)
    if prev is None:
        return pl.pallas_call(
            _tc_ln_body, grid=grid, in_specs=specs, out_specs=out_spec,
            out_shape=out_shape,
        )(g_s, pos_emb, tt_emb, t_f32, lnw, lnb)
    specs = [pl.BlockSpec(memory_space=pl.ANY)] + specs
    return pl.pallas_call(
        _tc_ln_body_prev, grid=grid, in_specs=specs, out_specs=out_spec,
        out_shape=out_shape, input_output_aliases={0: 0},
    )(prev, g_s, pos_emb, tt_emb, t_f32, lnw, lnb)


def kernel(input_ids, token_type_ids, word_embeddings, position_embeddings,
           token_type_embeddings, ln_weight, ln_bias):
    batch, seq = input_ids.shape
    hidden = word_embeddings.shape[1]
    n_tokens = batch * seq
    blk = seq // _NSPLIT
    ids = input_ids.astype(jnp.int32)
    t_f32 = token_type_ids.reshape(-1, 1).astype(jnp.float32)
    lnw = ln_weight.reshape(1, -1)
    lnb = ln_bias.reshape(1, -1)

    gathered = [
        _sc_gather(lax.slice(ids, (0, s * blk), (batch, (s + 1) * blk)).reshape(-1),
                   word_embeddings)
        for s in range(_NSPLIT)
    ]
    out = None
    for s in range(_NSPLIT):
        out = _tc_ln_slice(out, gathered[s], position_embeddings,
                           token_type_embeddings, t_f32, lnw, lnb, s, blk,
                           n_tokens, seq)
    return out.reshape(batch, seq, hidden)


# speedup vs baseline: 1.0989x; 1.0989x over previous
import functools

import jax
import jax.numpy as jnp
from jax import lax
from jax.experimental import pallas as pl
from jax.experimental.pallas import tpu as pltpu
from jax.experimental.pallas import tpu_sc as plsc

EPS = 1e-12

_NC = 2
_NS = 16
_NW = _NC * _NS

_CHUNK = 32
_NBUF = 4
_NSPLIT = 2


def _sc_gather_body(n_tokens, hidden, nbuf, ids_hbm, table_hbm, out_hbm, idx_v,
                    *bufs_and_sems):
    rows = bufs_and_sems[:nbuf]
    gsem = bufs_and_sems[nbuf:2 * nbuf]
    wsem = bufs_and_sems[2 * nbuf:]
    tok_per_w = n_tokens // _NW
    n_chunks = tok_per_w // _CHUNK
    wid = lax.axis_index("s") * _NC + lax.axis_index("c")
    base = wid * tok_per_w
    pltpu.sync_copy(ids_hbm.at[pl.ds(base, tok_per_w)], idx_v)

    def gather(c, b):
        return pltpu.async_copy(
            table_hbm.at[idx_v.at[pl.ds(c * _CHUNK, _CHUNK)]], rows[b], gsem[b])

    def writeback(c, b):
        return pltpu.async_copy(
            rows[b], out_hbm.at[pl.ds(base + c * _CHUNK, _CHUNK)], wsem[b])

    copies = [None] * nbuf
    wbs = [None] * nbuf
    for c in range(min(nbuf, n_chunks)):
        copies[c] = gather(c, c)
    for c in range(n_chunks):
        b = c % nbuf
        copies[b].wait()
        wbs[b] = writeback(c, b)
        nxt = c + nbuf
        if nxt < n_chunks:
            wbs[b].wait()
            copies[b] = gather(nxt, b)
    for c in range(max(0, n_chunks - nbuf), n_chunks):
        wbs[c % nbuf].wait()


def _sc_gather(ids_flat, table):
    n_tokens = ids_flat.shape[0]
    hidden = table.shape[1]
    tok_per_w = n_tokens // _NW
    nbuf = min(_NBUF, tok_per_w // _CHUNK)
    mesh = plsc.VectorSubcoreMesh(core_axis_name="c", subcore_axis_name="s")
    call = pl.kernel(
        functools.partial(_sc_gather_body, n_tokens, hidden, nbuf),
        mesh=mesh,
        out_type=jax.ShapeDtypeStruct((n_tokens, hidden), jnp.float32),
        scratch_types=[
            pltpu.VMEM((tok_per_w,), jnp.int32),
        ] + [pltpu.VMEM((_CHUNK, hidden), jnp.float32) for _ in range(nbuf)]
          + [pltpu.SemaphoreType.DMA for _ in range(2 * nbuf)],
    )
    return call(ids_flat, table)


def _tc_ln_body(x_ref, pos_ref, tt_ref, t_ref, w_ref, b_ref, o_ref):
    x = x_ref[...]
    tt0 = tt_ref[0:1, :]
    dtt = tt_ref[1:2, :] - tt0
    x = x + pos_ref[...] + tt0 + t_ref[...] * dtt
    u = jnp.mean(x, axis=-1, keepdims=True)
    v = jnp.mean((x - u) ** 2, axis=-1, keepdims=True)
    y = (x - u) * lax.rsqrt(v + EPS)
    o_ref[...] = y * w_ref[...] + b_ref[...]


def _tc_ln_body_prev(prev_ref, x_ref, pos_ref, tt_ref, t_ref, w_ref, b_ref, o_ref):
    del prev_ref
    _tc_ln_body(x_ref, pos_ref, tt_ref, t_ref, w_ref, b_ref, o_ref)


def _tc_ln_slice(prev, g_s, pos_emb, tt_emb, t_f32, lnw, lnb, s_idx, blk,
                 n_tokens, seq):
    hidden = g_s.shape[1]
    batch = n_tokens // seq
    n_sb = seq // blk
    grid = (batch,)
    specs = [
        pl.BlockSpec((blk, hidden), lambda b: (b, 0)),
        pl.BlockSpec((blk, hidden), lambda b: (s_idx, 0)),
        pl.BlockSpec((2, hidden), lambda b: (0, 0)),
        pl.BlockSpec((blk, 1), lambda b: (b * n_sb + s_idx, 0)),
        pl.BlockSpec((1, hidden), lambda b: (0, 0)),
        pl.BlockSpec((1, hidden), lambda b: (0, 0)),
    ]
    out_spec = pl.BlockSpec((blk, hidden), lambda b: (b * n_sb + s_idx, 0))
    out_shape = jax.ShapeDtypeStruct((n_tokens, hidden), jnp.float32)
    if prev is None:
        return pl.pallas_call(
            _tc_ln_body, grid=grid, in_specs=specs, out_specs=out_spec,
            out_shape=out_shape,
        )(g_s, pos_emb, tt_emb, t_f32, lnw, lnb)
    specs = [pl.BlockSpec(memory_space=pl.ANY)] + specs
    return pl.pallas_call(
        _tc_ln_body_prev, grid=grid, in_specs=specs, out_specs=out_spec,
        out_shape=out_shape, input_output_aliases={0: 0},
    )(prev, g_s, pos_emb, tt_emb, t_f32, lnw, lnb)


def kernel(input_ids, token_type_ids, word_embeddings, position_embeddings,
           token_type_embeddings, ln_weight, ln_bias):
    batch, seq = input_ids.shape
    hidden = word_embeddings.shape[1]
    n_tokens = batch * seq
    blk = seq // _NSPLIT
    ids = input_ids.astype(jnp.int32)
    t_f32 = token_type_ids.reshape(-1, 1).astype(jnp.float32)
    lnw = ln_weight.reshape(1, -1)
    lnb = ln_bias.reshape(1, -1)

    gathered = [
        _sc_gather(lax.slice(ids, (0, s * blk), (batch, (s + 1) * blk)).reshape(-1),
                   word_embeddings)
        for s in range(_NSPLIT)
    ]
    out = None
    for s in range(_NSPLIT):
        out = _tc_ln_slice(out, gathered[s], position_embeddings,
                           token_type_embeddings, t_f32, lnw, lnb, s, blk,
                           n_tokens, seq)
    return out.reshape(batch, seq, hidden)
